# baseline (speedup 1.0000x reference)
import functools

import jax
import jax.numpy as jnp
from jax import lax
from jax.experimental import pallas as pl
from jax.experimental.pallas import tpu as pltpu
from jax.experimental.pallas import tpu_sc as plsc

N = 10000
E = 320000
G = 64
D_IN = 128

NC = 2
NS = 16
CH = 128
_NCH0 = (E // NS + CH - 1) // CH
NCH = _NCH0 + (_NCH0 % 2)
EPAD = NS * NCH * CH
NA = 10240
RPW = NA // NS
TRASH = N
NCH_HALF = (NCH + 1) // 2

_f32 = jnp.float32
_sc_params = pltpu.CompilerParams(use_tc_tiling_on_sc=False)


def _sc_mesh():
  return plsc.VectorSubcoreMesh(
      core_axis_name="c", subcore_axis_name="s", num_cores=NC, num_subcores=NS)


def _zero_rows(buf, nrows, ncols):
  zero = jnp.zeros((16,), _f32)

  def body(i, _):
    for j in range(ncols // 16):
      buf[i, pl.ds(j * 16, 16)] = zero
    return 0

  lax.fori_loop(0, nrows, body, 0)


def _make_sc_edge_scatter(DH, NB):
  @functools.partial(
      pl.kernel,
      out_type=jax.ShapeDtypeStruct((NC, NA, DH), _f32),
      mesh=_sc_mesh(),
      scratch_types=[
          pltpu.VMEM((NCH, CH), jnp.int32),
          pltpu.VMEM((NCH, CH), jnp.int32),
      ] + [pltpu.VMEM((CH, DH), _f32)] * NB + [
          pltpu.VMEM_SHARED((NA, DH), _f32),
      ] + [pltpu.SemaphoreType.DMA] * (2 * NB),
      compiler_params=_sc_params,
  )
  def k(table_h, row_h, col_h, out_h, ridx, cidx, *rest):
    rbuf = rest[:NB]
    acc = rest[NB]
    gs = rest[NB + 1:NB + 1 + NB]
    ss = rest[NB + 1 + NB:]
    c = lax.axis_index("c")
    s = lax.axis_index("s")

    _zero_rows(rbuf[0], CH, DH)
    for z in range(RPW // CH):
      pltpu.sync_copy(rbuf[0], acc.at[pl.ds(s * RPW + z * CH, CH)])
    plsc.subcore_barrier()

    pltpu.sync_copy(row_h.at[s], ridx)
    pltpu.sync_copy(col_h.at[s], cidx)
    half = table_h.at[c]

    rbuf0, rbuf1 = rbuf[0], rbuf[1]
    gs0, gs1 = gs[0], gs[1]
    ss0, ss1 = ss[0], ss[1]
    _zero_rows(rbuf1, CH, DH)
    pltpu.async_copy(half.at[ridx.at[0]], rbuf0, gs0)
    pltpu.async_copy(rbuf1, acc.at[cidx.at[0]], ss1, add=True)

    def body(jj, _):
      j0 = 2 * jj
      j1 = j0 + 1
      pltpu.make_async_copy(half.at[ridx.at[j0]], rbuf0, gs0).wait()
      pltpu.async_copy(rbuf0, acc.at[cidx.at[j0]], ss0, add=True)
      pltpu.make_async_copy(rbuf1, acc.at[cidx.at[j1]], ss1).wait()
      pltpu.async_copy(half.at[ridx.at[j1]], rbuf1, gs1)
      pltpu.make_async_copy(half.at[ridx.at[j1]], rbuf1, gs1).wait()
      pltpu.async_copy(rbuf1, acc.at[cidx.at[j1]], ss1, add=True)
      pltpu.make_async_copy(rbuf0, acc.at[cidx.at[j0]], ss0).wait()
      j2 = lax.min(j0 + 2, NCH - 1)
      pltpu.async_copy(half.at[ridx.at[j2]], rbuf0, gs0)
      return 0

    lax.fori_loop(0, NCH // 2, body, 0)
    pltpu.make_async_copy(half.at[ridx.at[0]], rbuf0, gs0).wait()
    pltpu.make_async_copy(rbuf1, acc.at[cidx.at[0]], ss1).wait()
    plsc.subcore_barrier()

    pltpu.sync_copy(acc.at[pl.ds(s * RPW, RPW)], out_h.at[c, pl.ds(s * RPW, RPW)])

  return k


def _make_sc_degree():
  D = 16

  @functools.partial(
      pl.kernel,
      out_type=jax.ShapeDtypeStruct((NC, NA, D), _f32),
      mesh=_sc_mesh(),
      scratch_types=[
          pltpu.VMEM((NCH, CH), jnp.int32),
          pltpu.VMEM((CH, D), _f32),
          pltpu.VMEM_SHARED((NA, D), _f32),
      ],
      compiler_params=_sc_params,
  )
  def k(col_h, out_h, cidx, ones_buf, acc):
    c = lax.axis_index("c")
    s = lax.axis_index("s")

    _zero_rows(ones_buf, CH, D)
    for z in range(RPW // CH):
      pltpu.sync_copy(ones_buf, acc.at[pl.ds(s * RPW + z * CH, CH)])
    plsc.subcore_barrier()

    one = jnp.ones((16,), _f32)

    def fill(i, _):
      ones_buf[i, pl.ds(0, 16)] = one
      return 0

    lax.fori_loop(0, CH, fill, 0)

    pltpu.sync_copy(col_h.at[s], cidx)

    lo = c * NCH_HALF
    hi = lax.min(jnp.int32(NCH), lo + NCH_HALF)

    def body(j, _):
      pltpu.sync_copy(ones_buf, acc.at[cidx.at[j]], add=True)
      return 0

    lax.fori_loop(lo, hi, body, 0)
    plsc.subcore_barrier()

    pltpu.sync_copy(acc.at[pl.ds(s * RPW, RPW)], out_h.at[c, pl.ds(s * RPW, RPW)])

  return k


BN = 1000
GRID = N // BN


def _row_spec(d):
  return pl.BlockSpec((BN, d), lambda i: (i, 0))


def _half_spec(c, d):
  return pl.BlockSpec((1, BN, d), lambda i, c=c: (c, i, 0))


def _full_spec(*shape):
  ndim = len(shape)
  return pl.BlockSpec(shape, lambda i: (0,) * ndim)


def _split_store(o_ref, full, dh):
  o_ref[0] = full[:, 0:dh]
  o_ref[1] = full[:, dh:2 * dh]


def _stage_a_body(x_ref, w1_ref, aw1_ref, dp0_ref, dp1_ref, t1_ref, dinv_ref):
  deg = 1.0 + dp0_ref[0, :, 0:1] + dp1_ref[0, :, 0:1]
  dinv = lax.rsqrt(deg)
  dinv_ref[...] = dinv
  xb = x_ref[...]
  full = jnp.concatenate(
      [dinv * jnp.dot(xb, w1_ref[...], preferred_element_type=_f32),
       dinv * jnp.dot(xb, aw1_ref[...], preferred_element_type=_f32)], axis=1)
  _split_store(t1_ref, full, 96)


def _tc_stage_a(x, W1, aW1, degp):
  return pl.pallas_call(
      _stage_a_body,
      grid=(GRID,),
      in_specs=[
          _row_spec(D_IN),
          _full_spec(D_IN, 128),
          _full_spec(D_IN, 64),
          _half_spec(0, 16),
          _half_spec(1, 16),
      ],
      out_specs=[
          pl.BlockSpec((NC, BN, 96), lambda i: (0, i, 0)),
          _row_spec(1),
      ],
      out_shape=[
          jax.ShapeDtypeStruct((NC, N, 96), _f32),
          jax.ShapeDtypeStruct((N, 1), _f32),
      ],
  )(x, W1, aW1, degp, degp)


def _stage_bc_body(dh_in, dw_in, dh_out, p0_ref, p1_ref, t0_ref, t1_ref,
                   dinv_ref, bf_ref, bw_ref, wf_ref, ww_ref, o_ref):
  s = jnp.concatenate(
      [p0_ref[0] + t0_ref[0], p1_ref[0] + t1_ref[0]], axis=1)
  dinv = dinv_ref[...]
  f = jax.nn.relu(dinv * s[:, 0:128] + bf_ref[...])
  w = jax.nn.relu(dinv * s[:, 128:128 + dw_in] + bw_ref[...])
  full = jnp.concatenate(
      [dinv * jnp.dot(f, wf_ref[...], preferred_element_type=_f32),
       dinv * jnp.dot(w, ww_ref[...], preferred_element_type=_f32)], axis=1)
  _split_store(o_ref, full, dh_out)


def _tc_stage_bc(p, t, dinv, bf, bw, wf, ww, dh_in, dw_in, dw_out):
  dh_out = (128 + dw_out) // 2
  body = functools.partial(_stage_bc_body, dh_in, dw_in, dh_out)
  return pl.pallas_call(
      body,
      grid=(GRID,),
      in_specs=[
          _half_spec(0, dh_in),
          _half_spec(1, dh_in),
          _half_spec(0, dh_in),
          _half_spec(1, dh_in),
          _row_spec(1),
          _full_spec(1, 128),
          _full_spec(1, dw_in),
          _full_spec(128, 128),
          _full_spec(dw_in, dw_out),
      ],
      out_specs=pl.BlockSpec((NC, BN, dh_out), lambda i: (0, i, 0)),
      out_shape=jax.ShapeDtypeStruct((NC, N, dh_out), _f32),
  )(p, p, t, t, dinv, bf, bw, wf, ww)


def _stage_d1_body(p0_ref, p1_ref, t0_ref, t1_ref, dinv_ref, bf_ref, bw_ref,
                   f_ref, w_ref):
  s = jnp.concatenate(
      [p0_ref[0] + t0_ref[0], p1_ref[0] + t1_ref[0]], axis=1)
  dinv = dinv_ref[...]
  f_ref[...] = jax.nn.relu(dinv * s[:, 0:128] + bf_ref[...])
  w_ref[...] = jax.nn.relu(dinv * s[:, 128:129] + bw_ref[...])


def _tc_stage_d1(p, t, dinv, b3, ab3):
  return pl.pallas_call(
      _stage_d1_body,
      grid=(GRID,),
      in_specs=[
          _half_spec(0, 80),
          _half_spec(1, 80),
          _half_spec(0, 80),
          _half_spec(1, 80),
          _row_spec(1),
          _full_spec(1, 128),
          _full_spec(1, 1),
      ],
      out_specs=[_row_spec(128), _row_spec(1)],
      out_shape=[
          jax.ShapeDtypeStruct((N, 128), _f32),
          jax.ShapeDtypeStruct((N, 1), _f32),
      ],
  )(p, p, t, t, dinv, b3, ab3)


def _stage_d2_body(f_ref, w_ref, batch_ref, mw1_ref, mb1_ref, mw2_ref, mb2_ref,
                   o_ref):
  b = batch_ref[...]
  gid = lax.broadcasted_iota(jnp.int32, (N, G), 1)
  mask = jnp.where(gid == b, 1.0, 0.0)
  w = w_ref[...]
  neg = jnp.float32(-jnp.inf)
  masked = jnp.where(mask > 0.5, w, neg)
  m = jnp.max(masked, axis=0, keepdims=True)
  m = jnp.where(jnp.isfinite(m), m, 0.0)
  m_n = jnp.sum(mask * m, axis=1, keepdims=True)
  e = jnp.exp(w - m_n)
  dn0 = (((0,), (0,)), ((), ()))
  denom = lax.dot_general(e, mask, dn0, preferred_element_type=_f32)
  denom_n = jnp.sum(mask * denom, axis=1, keepdims=True)
  wsm = e / (denom_n + 1e-16)
  pooled = lax.dot_general(mask * wsm, f_ref[...], dn0,
                           preferred_element_type=_f32)
  h = jax.nn.relu(
      jnp.dot(pooled, mw1_ref[...], preferred_element_type=_f32) + mb1_ref[...])
  o_ref[...] = jnp.dot(h, mw2_ref[...], preferred_element_type=_f32) + mb2_ref[...]


def _tc_stage_d2(f, w, batch2d, mW1, mb1, mW2, mb2):
  return pl.pallas_call(
      _stage_d2_body,
      out_shape=jax.ShapeDtypeStruct((G, 256), _f32),
  )(f, w, batch2d, mW1, mb1, mW2, mb2)


def kernel(x, edge_index, batch, W1, b1, W2, b2, W3, b3, aW1, ab1, aW2, ab2,
           aW3, ab3, mW1, mb1, mW2, mb2):
  pad = EPAD - E
  row = jnp.concatenate([edge_index[0], jnp.zeros((pad,), jnp.int32)])
  col = jnp.concatenate([edge_index[1], jnp.full((pad,), TRASH, jnp.int32)])
  row_r = row.reshape(NS, NCH, CH)
  col_r = col.reshape(NS, NCH, CH)
  batch2d = batch.reshape(N, 1)
  aW3p = jnp.pad(aW3, ((0, 0), (0, 31)))
  b1r, b2r, b3r = b1.reshape(1, -1), b2.reshape(1, -1), b3.reshape(1, -1)
  ab1r, ab2r = ab1.reshape(1, -1), ab2.reshape(1, -1)
  ab3r = ab3.reshape(1, 1)
  mb1r, mb2r = mb1.reshape(1, -1), mb2.reshape(1, -1)

  degp = _make_sc_degree()(col_r)

  t1, dinv = _tc_stage_a(x, W1, aW1, degp)

  p1 = _make_sc_edge_scatter(96, 2)(t1, row_r, col_r)
  t2 = _tc_stage_bc(p1, t1, dinv, b1r, ab1r, W2, aW2, 96, 64, 32)
  p2 = _make_sc_edge_scatter(80, 2)(t2, row_r, col_r)
  t3 = _tc_stage_bc(p2, t2, dinv, b2r, ab2r, W3, aW3p, 80, 32, 32)
  p3 = _make_sc_edge_scatter(80, 2)(t3, row_r, col_r)

  f3, w3 = _tc_stage_d1(p3, t3, dinv, b3r, ab3r)
  return _tc_stage_d2(f3, w3, batch2d, mW1, mb1r, mW2, mb2r)


# speedup vs baseline: 1.1363x; 1.1363x over previous
import functools

import jax
import jax.numpy as jnp
from jax import lax
from jax.experimental import pallas as pl
from jax.experimental.pallas import tpu as pltpu
from jax.experimental.pallas import tpu_sc as plsc

N = 10000
E = 320000
G = 64
D_IN = 128

NC = 2
NS = 16
CH = 128
_NCH0 = (E // NS + CH - 1) // CH
NCH = _NCH0 + (_NCH0 % 2)
EPAD = NS * NCH * CH
NA = 10240
RPW = NA // NS
TRASH = N
NCH_HALF = (NCH + 1) // 2

_f32 = jnp.float32
_sc_params = pltpu.CompilerParams(use_tc_tiling_on_sc=False)


def _sc_mesh():
  return plsc.VectorSubcoreMesh(
      core_axis_name="c", subcore_axis_name="s", num_cores=NC, num_subcores=NS)


def _zero_rows(buf, nrows, ncols):
  zero = jnp.zeros((16,), _f32)

  def body(i, _):
    for j in range(ncols // 16):
      buf[i, pl.ds(j * 16, 16)] = zero
    return 0

  lax.fori_loop(0, nrows, body, 0)


def _make_sc_edge_scatter(DH, NB):
  @functools.partial(
      pl.kernel,
      out_type=jax.ShapeDtypeStruct((NC, NA, DH), _f32),
      mesh=_sc_mesh(),
      scratch_types=[
          pltpu.VMEM((NCH, CH), jnp.int32),
          pltpu.VMEM((NCH, CH), jnp.int32),
      ] + [pltpu.VMEM((CH, DH), _f32)] * NB + [
          pltpu.VMEM_SHARED((NA, DH), _f32),
      ] + [pltpu.SemaphoreType.DMA] * NB,
      compiler_params=_sc_params,
  )
  def k(table_h, row_h, col_h, out_h, ridx, cidx, *rest):
    rbuf = rest[:NB]
    acc = rest[NB]
    gs = rest[NB + 1:]
    c = lax.axis_index("c")
    s = lax.axis_index("s")

    _zero_rows(rbuf[0], CH, DH)
    for z in range(RPW // CH):
      pltpu.sync_copy(rbuf[0], acc.at[pl.ds(s * RPW + z * CH, CH)])
    plsc.subcore_barrier()

    pltpu.sync_copy(row_h.at[s], ridx)
    pltpu.sync_copy(col_h.at[s], cidx)
    half = table_h.at[c]

    for b in range(NB - 1):
      pltpu.async_copy(half.at[ridx.at[b]], rbuf[b], gs[b])

    def body(jj, _):
      base = NB * jj
      jtop = lax.min(base + NB - 1, NCH - 1)
      pltpu.async_copy(half.at[ridx.at[jtop]], rbuf[NB - 1], gs[NB - 1])
      for b in range(NB):
        j = base + b
        pltpu.make_async_copy(half.at[ridx.at[j]], rbuf[b], gs[b]).wait()
        pltpu.sync_copy(rbuf[b], acc.at[cidx.at[j]], add=True)
        if b < NB - 1:
          jn = lax.min(j + NB, NCH - 1)
          pltpu.async_copy(half.at[ridx.at[jn]], rbuf[b], gs[b])
      return 0

    lax.fori_loop(0, NCH // NB, body, 0)
    for b in range(NB - 1):
      pltpu.make_async_copy(half.at[ridx.at[0]], rbuf[b], gs[b]).wait()
    plsc.subcore_barrier()

    pltpu.sync_copy(acc.at[pl.ds(s * RPW, RPW)], out_h.at[c, pl.ds(s * RPW, RPW)])

  return k


def _make_sc_degree():
  D = 16

  @functools.partial(
      pl.kernel,
      out_type=jax.ShapeDtypeStruct((NC, NA, D), _f32),
      mesh=_sc_mesh(),
      scratch_types=[
          pltpu.VMEM((NCH, CH), jnp.int32),
          pltpu.VMEM((CH, D), _f32),
          pltpu.VMEM_SHARED((NA, D), _f32),
      ],
      compiler_params=_sc_params,
  )
  def k(col_h, out_h, cidx, ones_buf, acc):
    c = lax.axis_index("c")
    s = lax.axis_index("s")

    _zero_rows(ones_buf, CH, D)
    for z in range(RPW // CH):
      pltpu.sync_copy(ones_buf, acc.at[pl.ds(s * RPW + z * CH, CH)])
    plsc.subcore_barrier()

    one = jnp.ones((16,), _f32)

    def fill(i, _):
      ones_buf[i, pl.ds(0, 16)] = one
      return 0

    lax.fori_loop(0, CH, fill, 0)

    pltpu.sync_copy(col_h.at[s], cidx)

    lo = c * NCH_HALF
    hi = lax.min(jnp.int32(NCH), lo + NCH_HALF)

    def body(j, _):
      pltpu.sync_copy(ones_buf, acc.at[cidx.at[j]], add=True)
      return 0

    lax.fori_loop(lo, hi, body, 0)
    plsc.subcore_barrier()

    pltpu.sync_copy(acc.at[pl.ds(s * RPW, RPW)], out_h.at[c, pl.ds(s * RPW, RPW)])

  return k


BN = 1000
GRID = N // BN


def _row_spec(d):
  return pl.BlockSpec((BN, d), lambda i: (i, 0))


def _half_spec(c, d):
  return pl.BlockSpec((1, BN, d), lambda i, c=c: (c, i, 0))


def _full_spec(*shape):
  ndim = len(shape)
  return pl.BlockSpec(shape, lambda i: (0,) * ndim)


def _split_store(o_ref, full, dh):
  o_ref[0] = full[:, 0:dh]
  o_ref[1] = full[:, dh:2 * dh]


def _stage_a_body(x_ref, w1_ref, aw1_ref, dp0_ref, dp1_ref, t1_ref, dinv_ref):
  deg = 1.0 + dp0_ref[0, :, 0:1] + dp1_ref[0, :, 0:1]
  dinv = lax.rsqrt(deg)
  dinv_ref[...] = dinv
  xb = x_ref[...]
  full = jnp.concatenate(
      [dinv * jnp.dot(xb, w1_ref[...], preferred_element_type=_f32),
       dinv * jnp.dot(xb, aw1_ref[...], preferred_element_type=_f32)], axis=1)
  _split_store(t1_ref, full, 96)


def _tc_stage_a(x, W1, aW1, degp):
  return pl.pallas_call(
      _stage_a_body,
      grid=(GRID,),
      in_specs=[
          _row_spec(D_IN),
          _full_spec(D_IN, 128),
          _full_spec(D_IN, 64),
          _half_spec(0, 16),
          _half_spec(1, 16),
      ],
      out_specs=[
          pl.BlockSpec((NC, BN, 96), lambda i: (0, i, 0)),
          _row_spec(1),
      ],
      out_shape=[
          jax.ShapeDtypeStruct((NC, N, 96), _f32),
          jax.ShapeDtypeStruct((N, 1), _f32),
      ],
  )(x, W1, aW1, degp, degp)


def _stage_bc_body(dh_in, dw_in, dh_out, p0_ref, p1_ref, t0_ref, t1_ref,
                   dinv_ref, bf_ref, bw_ref, wf_ref, ww_ref, o_ref):
  s = jnp.concatenate(
      [p0_ref[0] + t0_ref[0], p1_ref[0] + t1_ref[0]], axis=1)
  dinv = dinv_ref[...]
  f = jax.nn.relu(dinv * s[:, 0:128] + bf_ref[...])
  w = jax.nn.relu(dinv * s[:, 128:128 + dw_in] + bw_ref[...])
  full = jnp.concatenate(
      [dinv * jnp.dot(f, wf_ref[...], preferred_element_type=_f32),
       dinv * jnp.dot(w, ww_ref[...], preferred_element_type=_f32)], axis=1)
  _split_store(o_ref, full, dh_out)


def _tc_stage_bc(p, t, dinv, bf, bw, wf, ww, dh_in, dw_in, dw_out):
  dh_out = (128 + dw_out) // 2
  body = functools.partial(_stage_bc_body, dh_in, dw_in, dh_out)
  return pl.pallas_call(
      body,
      grid=(GRID,),
      in_specs=[
          _half_spec(0, dh_in),
          _half_spec(1, dh_in),
          _half_spec(0, dh_in),
          _half_spec(1, dh_in),
          _row_spec(1),
          _full_spec(1, 128),
          _full_spec(1, dw_in),
          _full_spec(128, 128),
          _full_spec(dw_in, dw_out),
      ],
      out_specs=pl.BlockSpec((NC, BN, dh_out), lambda i: (0, i, 0)),
      out_shape=jax.ShapeDtypeStruct((NC, N, dh_out), _f32),
  )(p, p, t, t, dinv, bf, bw, wf, ww)


def _stage_d1_body(p0_ref, p1_ref, t0_ref, t1_ref, dinv_ref, bf_ref, bw_ref,
                   f_ref, w_ref):
  s = jnp.concatenate(
      [p0_ref[0] + t0_ref[0], p1_ref[0] + t1_ref[0]], axis=1)
  dinv = dinv_ref[...]
  f_ref[...] = jax.nn.relu(dinv * s[:, 0:128] + bf_ref[...])
  w_ref[...] = jax.nn.relu(dinv * s[:, 128:129] + bw_ref[...])


def _tc_stage_d1(p, t, dinv, b3, ab3):
  return pl.pallas_call(
      _stage_d1_body,
      grid=(GRID,),
      in_specs=[
          _half_spec(0, 80),
          _half_spec(1, 80),
          _half_spec(0, 80),
          _half_spec(1, 80),
          _row_spec(1),
          _full_spec(1, 128),
          _full_spec(1, 1),
      ],
      out_specs=[_row_spec(128), _row_spec(1)],
      out_shape=[
          jax.ShapeDtypeStruct((N, 128), _f32),
          jax.ShapeDtypeStruct((N, 1), _f32),
      ],
  )(p, p, t, t, dinv, b3, ab3)


def _stage_d2_body(f_ref, w_ref, batch_ref, mw1_ref, mb1_ref, mw2_ref, mb2_ref,
                   o_ref):
  b = batch_ref[...]
  gid = lax.broadcasted_iota(jnp.int32, (N, G), 1)
  mask = jnp.where(gid == b, 1.0, 0.0)
  w = w_ref[...]
  neg = jnp.float32(-jnp.inf)
  masked = jnp.where(mask > 0.5, w, neg)
  m = jnp.max(masked, axis=0, keepdims=True)
  m = jnp.where(jnp.isfinite(m), m, 0.0)
  m_n = jnp.sum(mask * m, axis=1, keepdims=True)
  e = jnp.exp(w - m_n)
  dn0 = (((0,), (0,)), ((), ()))
  denom = lax.dot_general(e, mask, dn0, preferred_element_type=_f32)
  denom_n = jnp.sum(mask * denom, axis=1, keepdims=True)
  wsm = e / (denom_n + 1e-16)
  pooled = lax.dot_general(mask * wsm, f_ref[...], dn0,
                           preferred_element_type=_f32)
  h = jax.nn.relu(
      jnp.dot(pooled, mw1_ref[...], preferred_element_type=_f32) + mb1_ref[...])
  o_ref[...] = jnp.dot(h, mw2_ref[...], preferred_element_type=_f32) + mb2_ref[...]


def _tc_stage_d2(f, w, batch2d, mW1, mb1, mW2, mb2):
  return pl.pallas_call(
      _stage_d2_body,
      out_shape=jax.ShapeDtypeStruct((G, 256), _f32),
  )(f, w, batch2d, mW1, mb1, mW2, mb2)


def kernel(x, edge_index, batch, W1, b1, W2, b2, W3, b3, aW1, ab1, aW2, ab2,
           aW3, ab3, mW1, mb1, mW2, mb2):
  pad = EPAD - E
  row = jnp.concatenate([edge_index[0], jnp.zeros((pad,), jnp.int32)])
  col = jnp.concatenate([edge_index[1], jnp.full((pad,), TRASH, jnp.int32)])
  row_r = row.reshape(NS, NCH, CH)
  col_r = col.reshape(NS, NCH, CH)
  batch2d = batch.reshape(N, 1)
  aW3p = jnp.pad(aW3, ((0, 0), (0, 31)))
  b1r, b2r, b3r = b1.reshape(1, -1), b2.reshape(1, -1), b3.reshape(1, -1)
  ab1r, ab2r = ab1.reshape(1, -1), ab2.reshape(1, -1)
  ab3r = ab3.reshape(1, 1)
  mb1r, mb2r = mb1.reshape(1, -1), mb2.reshape(1, -1)

  degp = _make_sc_degree()(col_r)

  t1, dinv = _tc_stage_a(x, W1, aW1, degp)

  p1 = _make_sc_edge_scatter(96, 2)(t1, row_r, col_r)
  t2 = _tc_stage_bc(p1, t1, dinv, b1r, ab1r, W2, aW2, 96, 64, 32)
  p2 = _make_sc_edge_scatter(80, 2)(t2, row_r, col_r)
  t3 = _tc_stage_bc(p2, t2, dinv, b2r, ab2r, W3, aW3p, 80, 32, 32)
  p3 = _make_sc_edge_scatter(80, 2)(t3, row_r, col_r)

  f3, w3 = _tc_stage_d1(p3, t3, dinv, b3r, ab3r)
  return _tc_stage_d2(f3, w3, batch2d, mW1, mb1r, mW2, mb2r)
